# baseline (speedup 1.0000x reference)
import jax
import jax.numpy as jnp
from jax import lax
from jax.experimental import pallas as pl
from jax.experimental.pallas import tpu as pltpu
from jax.experimental.pallas import tpu_sc as plsc

ENT_NUM = 100000
REL_NUM = 1000
D = 128
BATCH = 4096
NEG_NUM = 20
MARGIN = 1.0

N_TRIPLES = BATCH + BATCH + BATCH * NEG_NUM
NC, NS = 2, 16
NW = NC * NS
PER_W = N_TRIPLES // NW
C = 88
NCHUNK = PER_W // C
NV = D // 16
DOT_BLK = 8192


def _tree_sum(vals):
    vals = list(vals)
    while len(vals) > 1:
        nxt = [vals[k] + vals[k + 1] for k in range(0, len(vals) - 1, 2)]
        if len(vals) % 2:
            nxt.append(vals[-1])
        vals = nxt
    return vals[0]


def _rsqrt_s(x):
    xh = x * 0.5
    i = lax.bitcast_convert_type(x, jnp.int32)
    i = jnp.int32(0x5F3759DF) - lax.shift_right_logical(i, 1)
    y = lax.bitcast_convert_type(i, jnp.float32)
    y = y * (1.5 - xh * y * y)
    y = y * (1.5 - xh * y * y)
    y = y * (1.5 - xh * y * y)
    return y


def _dots_body(e_ref, t_ref, o_ref):
    d = jnp.sum(e_ref[...] * t_ref[...], axis=1)
    o_ref[...] = d


def _ent_dots(ent_e, ent_t):
    return pl.pallas_call(
        _dots_body,
        grid=(pl.cdiv(ENT_NUM, DOT_BLK),),
        in_specs=[
            pl.BlockSpec((DOT_BLK, D), lambda i: (i, 0)),
            pl.BlockSpec((DOT_BLK, D), lambda i: (i, 0)),
        ],
        out_specs=pl.BlockSpec((DOT_BLK,), lambda i: (i,)),
        out_shape=jax.ShapeDtypeStruct((ENT_NUM,), jnp.float32),
    )(ent_e, ent_t)


def _reln_body(r_ref, o_ref):
    r = r_ref[...]
    sq = jnp.sum(r * r, axis=1, keepdims=True)
    o_ref[...] = r * lax.rsqrt(jnp.maximum(sq, 1e-12))


def _rel_norm(rel_e):
    return pl.pallas_call(
        _reln_body,
        out_shape=jax.ShapeDtypeStruct((REL_NUM, D), jnp.float32),
    )(rel_e)


def _sc_body(ent_e, rel_n, rel_t, dots, h_hbm, t_hbm, r_hbm, oi_hbm, out,
             hv, tv, rv, ov, bufs, dbufs, scores_v, sem_a, sem_b):
    sems = (sem_a, sem_b)
    wid = lax.axis_index("s") * NC + lax.axis_index("c")
    base = wid * PER_W
    pltpu.sync_copy(h_hbm.at[pl.ds(base, PER_W)], hv)
    pltpu.sync_copy(t_hbm.at[pl.ds(base, PER_W)], tv)
    pltpu.sync_copy(r_hbm.at[pl.ds(base, PER_W)], rv)
    pltpu.sync_copy(oi_hbm.at[wid], ov)

    def copies(g, slot):
        hi = hv.at[pl.ds(g * C, C)]
        ti = tv.at[pl.ds(g * C, C)]
        ri = rv.at[pl.ds(g * C, C)]
        sem = sems[slot]
        return (
            pltpu.make_async_copy(ent_e.at[hi], bufs.at[slot, 0], sem),
            pltpu.make_async_copy(ent_e.at[ti], bufs.at[slot, 1], sem),
            pltpu.make_async_copy(rel_n.at[ri], bufs.at[slot, 2], sem),
            pltpu.make_async_copy(rel_t.at[ri], bufs.at[slot, 3], sem),
            pltpu.make_async_copy(dots.at[hi], dbufs.at[slot, 0, pl.ds(0, C)],
                                  sem),
            pltpu.make_async_copy(dots.at[ti], dbufs.at[slot, 1, pl.ds(0, C)],
                                  sem),
        )

    def fire(g, slot):
        for cp in copies(g, slot):
            cp.start()

    def drain(g, slot):
        for cp in copies(g, slot):
            cp.wait()

    def compute(g, slot):
        hb = bufs.at[slot, 0]
        tb = bufs.at[slot, 1]
        rnb = bufs.at[slot, 2]
        rtb = bufs.at[slot, 3]
        dhb = dbufs.at[slot, 0]
        dtb = dbufs.at[slot, 1]
        last_lane = lax.iota(jnp.int32, 16) == 15

        def tri(i):
            h = [hb[i, pl.ds(16 * j, 16)] for j in range(NV)]
            t = [tb[i, pl.ds(16 * j, 16)] for j in range(NV)]
            rn = [rnb[i, pl.ds(16 * j, 16)] for j in range(NV)]
            rt = [rtb[i, pl.ds(16 * j, 16)] for j in range(NV)]
            dhs = dhb[pl.ds(i, 16)][0]
            dts = dtb[pl.ds(i, 16)][0]

            hp = [h[j] + dhs * rt[j] for j in range(NV)]
            tp = [t[j] + dts * rt[j] for j in range(NV)]
            nh = jnp.sum(_tree_sum([hp[j] * hp[j] for j in range(NV)]))
            nt = jnp.sum(_tree_sum([tp[j] * tp[j] for j in range(NV)]))

            inh = _rsqrt_s(jnp.maximum(nh, 1e-12))
            int_ = _rsqrt_s(jnp.maximum(nt, 1e-12))

            s_acc = _tree_sum([jnp.abs(hp[j] * inh + rn[j] - tp[j] * int_)
                               for j in range(NV)])
            cs = plsc.cumsum(s_acc)
            plsc.store_compressed(scores_v.at[pl.ds(g * C + i, 16)], cs,
                                  mask=last_lane)

        plsc.parallel_loop(0, C, unroll=2)(tri)

    fire(0, 0)
    fire(1, 1)

    def ring(k, _):
        g0 = k * 2
        for b in range(2):
            g = g0 + b
            drain(g, b)
            compute(g, b)

            @pl.when(g + 2 < NCHUNK)
            def _():
                fire(g + 2, b)
        return 0

    lax.fori_loop(0, NCHUNK // 2, ring, 0)
    pltpu.async_copy(scores_v.at[pl.ds(0, PER_W)], out.at[ov], sem_a).wait()


_sc_scores = pl.kernel(
    _sc_body,
    out_type=jax.ShapeDtypeStruct((N_TRIPLES,), jnp.float32),
    mesh=plsc.VectorSubcoreMesh(core_axis_name="c", subcore_axis_name="s"),
    compiler_params=pltpu.CompilerParams(needs_layout_passes=False),
    scratch_types=[
        pltpu.VMEM((PER_W,), jnp.int32),
        pltpu.VMEM((PER_W,), jnp.int32),
        pltpu.VMEM((PER_W,), jnp.int32),
        pltpu.VMEM((PER_W,), jnp.int32),
        pltpu.VMEM((2, 4, C, D), jnp.float32),
        pltpu.VMEM((2, 2, C + 16), jnp.float32),
        pltpu.VMEM((PER_W + 16,), jnp.float32),
        pltpu.SemaphoreType.DMA,
        pltpu.SemaphoreType.DMA,
    ],
)


def _tail_body(ps_ref, ns_ref, nss_ref, sim_ref, loss_ref, negloss_ref):
    ps = ps_ref[...]
    ns = ns_ref[...]
    loss_ref[...] = jnp.sum(jnp.maximum(ps - ns + MARGIN, 0.0),
                            axis=(0, 1), keepdims=True)
    nss = nss_ref[...]
    sim = sim_ref[...]
    a = jax.nn.softmax(nss, axis=0)
    b = jax.nn.softmax(sim, axis=-1)
    ab = jnp.dot(a, b, preferred_element_type=jnp.float32)
    eye = (lax.broadcasted_iota(jnp.int32, (NEG_NUM, NEG_NUM), 0) ==
           lax.broadcasted_iota(jnp.int32, (NEG_NUM, NEG_NUM), 1))
    negloss_ref[...] = jnp.sum(jnp.where(eye, ab, 0.0),
                               axis=(0, 1), keepdims=True) / BATCH


def _tail(ps, ns, nss, sim):
    return pl.pallas_call(
        _tail_body,
        out_shape=[
            jax.ShapeDtypeStruct((1, 1), jnp.float32),
            jax.ShapeDtypeStruct((1, 1), jnp.float32),
        ],
    )(ps, ns, nss, sim)


def kernel(ent_embeddings, rel_embeddings, ent_transfer, rel_transfer,
           pos_h, pos_t, pos_r, neg_h, neg_t, neg_r,
           neg_hs, neg_ts, neg_rs, neg_sim):
    i32 = jnp.int32
    H = jnp.concatenate([pos_h.astype(i32), neg_h.astype(i32),
                         neg_hs.astype(i32).reshape(-1)])
    T = jnp.concatenate([pos_t.astype(i32), neg_t.astype(i32),
                         neg_ts.astype(i32).reshape(-1)])
    R = jnp.concatenate([pos_r.astype(i32), neg_r.astype(i32),
                         neg_rs.astype(i32).reshape(-1)])
    dots = _ent_dots(ent_embeddings, ent_transfer)
    rel_n = _rel_norm(rel_embeddings)
    perm_ns = (jnp.arange(NEG_NUM, dtype=i32)[None, :] * BATCH
               + jnp.arange(BATCH, dtype=i32)[:, None]).reshape(-1)
    out_idx = jnp.concatenate([jnp.arange(2 * BATCH, dtype=i32),
                               2 * BATCH + perm_ns]).reshape(NW, PER_W)
    scores = _sc_scores(ent_embeddings, rel_n, rel_transfer, dots,
                        H, T, R, out_idx)
    ps = scores[:BATCH]
    ns = scores[BATCH:2 * BATCH]
    nss = scores[2 * BATCH:].reshape(NEG_NUM, BATCH)
    loss2, negloss2 = _tail(ps.reshape(BATCH // D, D),
                            ns.reshape(BATCH // D, D), nss, neg_sim)
    return loss2[0, 0], negloss2[0, 0], ps


# speedup vs baseline: 1.7677x; 1.7677x over previous
import jax
import jax.numpy as jnp
from jax import lax
from jax.experimental import pallas as pl
from jax.experimental.pallas import tpu as pltpu
from jax.experimental.pallas import tpu_sc as plsc

ENT_NUM = 100000
REL_NUM = 1000
D = 128
BATCH = 4096
NEG_NUM = 20
MARGIN = 1.0

N_TRIPLES = BATCH + BATCH + BATCH * NEG_NUM
NC, NS = 2, 16
NW = NC * NS
PER_W = N_TRIPLES // NW
C = 88
NCHUNK = PER_W // C
NV = D // 16
DOT_BLK = 8192


def _tree_sum(vals):
    vals = list(vals)
    while len(vals) > 1:
        nxt = [vals[k] + vals[k + 1] for k in range(0, len(vals) - 1, 2)]
        if len(vals) % 2:
            nxt.append(vals[-1])
        vals = nxt
    return vals[0]


def _rsqrt_s(x):
    xh = x * 0.5
    i = lax.bitcast_convert_type(x, jnp.int32)
    i = jnp.int32(0x5F3759DF) - lax.shift_right_logical(i, 1)
    y = lax.bitcast_convert_type(i, jnp.float32)
    y = y * (1.5 - xh * y * y)
    y = y * (1.5 - xh * y * y)
    y = y * (1.5 - xh * y * y)
    return y


def _dots_body(e_ref, t_ref, o_ref):
    d = jnp.sum(e_ref[...] * t_ref[...], axis=1)
    o_ref[...] = d


def _ent_dots(ent_e, ent_t):
    return pl.pallas_call(
        _dots_body,
        grid=(pl.cdiv(ENT_NUM, DOT_BLK),),
        in_specs=[
            pl.BlockSpec((DOT_BLK, D), lambda i: (i, 0)),
            pl.BlockSpec((DOT_BLK, D), lambda i: (i, 0)),
        ],
        out_specs=pl.BlockSpec((DOT_BLK,), lambda i: (i,)),
        out_shape=jax.ShapeDtypeStruct((ENT_NUM,), jnp.float32),
    )(ent_e, ent_t)


def _reln_body(r_ref, o_ref):
    r = r_ref[...]
    sq = jnp.sum(r * r, axis=1, keepdims=True)
    o_ref[...] = r * lax.rsqrt(jnp.maximum(sq, 1e-12))


def _rel_norm(rel_e):
    return pl.pallas_call(
        _reln_body,
        out_shape=jax.ShapeDtypeStruct((REL_NUM, D), jnp.float32),
    )(rel_e)


N1 = 36864
N2 = N_TRIPLES - N1
A_PER_W = N1 // NW
A_C = 64
A_NCHUNK = A_PER_W // A_C
B_PER_W = N2 // NW
B_C = 64
B_NCHUNK = B_PER_W // B_C


def _sc_body_a(ent_e, rel_e, ent_t, rel_t, h_hbm, t_hbm, r_hbm, out,
               hv, tv, rv, bufs, scores_v, sem_a, sem_b):
    sems = (sem_a, sem_b)
    wid = lax.axis_index("s") * NC + lax.axis_index("c")
    base = wid * A_PER_W
    pltpu.sync_copy(h_hbm.at[pl.ds(base, A_PER_W)], hv)
    pltpu.sync_copy(t_hbm.at[pl.ds(base, A_PER_W)], tv)
    pltpu.sync_copy(r_hbm.at[pl.ds(base, A_PER_W)], rv)

    def copies(g, slot):
        hi = hv.at[pl.ds(g * A_C, A_C)]
        ti = tv.at[pl.ds(g * A_C, A_C)]
        ri = rv.at[pl.ds(g * A_C, A_C)]
        sem = sems[slot]
        return (
            pltpu.make_async_copy(ent_e.at[hi], bufs.at[slot, 0], sem),
            pltpu.make_async_copy(ent_t.at[hi], bufs.at[slot, 1], sem),
            pltpu.make_async_copy(ent_e.at[ti], bufs.at[slot, 2], sem),
            pltpu.make_async_copy(ent_t.at[ti], bufs.at[slot, 3], sem),
            pltpu.make_async_copy(rel_e.at[ri], bufs.at[slot, 4], sem),
            pltpu.make_async_copy(rel_t.at[ri], bufs.at[slot, 5], sem),
        )

    def fire(g, slot):
        for cp in copies(g, slot):
            cp.start()

    def drain(g, slot):
        for cp in copies(g, slot):
            cp.wait()

    def compute(g, slot):
        hb = bufs.at[slot, 0]
        htb = bufs.at[slot, 1]
        tb = bufs.at[slot, 2]
        ttb = bufs.at[slot, 3]
        rb = bufs.at[slot, 4]
        rtb = bufs.at[slot, 5]
        last_lane = lax.iota(jnp.int32, 16) == 15

        def tri(i):
            h = [hb[i, pl.ds(16 * j, 16)] for j in range(NV)]
            ht = [htb[i, pl.ds(16 * j, 16)] for j in range(NV)]
            t = [tb[i, pl.ds(16 * j, 16)] for j in range(NV)]
            tt = [ttb[i, pl.ds(16 * j, 16)] for j in range(NV)]
            r = [rb[i, pl.ds(16 * j, 16)] for j in range(NV)]
            rt = [rtb[i, pl.ds(16 * j, 16)] for j in range(NV)]

            dh = jnp.sum(_tree_sum([h[j] * ht[j] for j in range(NV)]))
            dt = jnp.sum(_tree_sum([t[j] * tt[j] for j in range(NV)]))
            nr = jnp.sum(_tree_sum([r[j] * r[j] for j in range(NV)]))

            hp = [h[j] + dh * rt[j] for j in range(NV)]
            tp = [t[j] + dt * rt[j] for j in range(NV)]
            nh = jnp.sum(_tree_sum([hp[j] * hp[j] for j in range(NV)]))
            nt = jnp.sum(_tree_sum([tp[j] * tp[j] for j in range(NV)]))

            inh = _rsqrt_s(jnp.maximum(nh, 1e-12))
            int_ = _rsqrt_s(jnp.maximum(nt, 1e-12))
            inr = _rsqrt_s(jnp.maximum(nr, 1e-12))

            s_acc = _tree_sum([jnp.abs(hp[j] * inh + r[j] * inr
                                       - tp[j] * int_) for j in range(NV)])
            cs = plsc.cumsum(s_acc)
            plsc.store_compressed(scores_v.at[pl.ds(g * A_C + i, 16)], cs,
                                  mask=last_lane)

        plsc.parallel_loop(0, A_C, unroll=2)(tri)

    fire(0, 0)
    fire(1, 1)

    def ring(k, _):
        g0 = k * 2
        for b in range(2):
            g = g0 + b
            drain(g, b)
            compute(g, b)

            @pl.when(g + 2 < A_NCHUNK)
            def _():
                fire(g + 2, b)
        return 0

    lax.fori_loop(0, A_NCHUNK // 2, ring, 0)
    pltpu.sync_copy(scores_v.at[pl.ds(0, A_PER_W)],
                    out.at[pl.ds(base, A_PER_W)])


_sc_scores_a = pl.kernel(
    _sc_body_a,
    out_type=jax.ShapeDtypeStruct((N1,), jnp.float32),
    mesh=plsc.VectorSubcoreMesh(core_axis_name="c", subcore_axis_name="s"),
    compiler_params=pltpu.CompilerParams(needs_layout_passes=False),
    scratch_types=[
        pltpu.VMEM((A_PER_W,), jnp.int32),
        pltpu.VMEM((A_PER_W,), jnp.int32),
        pltpu.VMEM((A_PER_W,), jnp.int32),
        pltpu.VMEM((2, 6, A_C, D), jnp.float32),
        pltpu.VMEM((A_PER_W + 16,), jnp.float32),
        pltpu.SemaphoreType.DMA,
        pltpu.SemaphoreType.DMA,
    ],
)

def _sc_body(ent_e, rel_n, rel_t, dots, h_hbm, t_hbm, r_hbm, out,
             hv, tv, rv, bufs, dbufs, scores_v, sem_a, sem_b):
    sems = (sem_a, sem_b)
    wid = lax.axis_index("s") * NC + lax.axis_index("c")
    base = wid * B_PER_W
    pltpu.sync_copy(h_hbm.at[pl.ds(base, B_PER_W)], hv)
    pltpu.sync_copy(t_hbm.at[pl.ds(base, B_PER_W)], tv)
    pltpu.sync_copy(r_hbm.at[pl.ds(base, B_PER_W)], rv)

    def copies(g, slot):
        hi = hv.at[pl.ds(g * B_C, B_C)]
        ti = tv.at[pl.ds(g * B_C, B_C)]
        ri = rv.at[pl.ds(g * B_C, B_C)]
        sem = sems[slot]
        return (
            pltpu.make_async_copy(ent_e.at[hi], bufs.at[slot, 0], sem),
            pltpu.make_async_copy(ent_e.at[ti], bufs.at[slot, 1], sem),
            pltpu.make_async_copy(rel_n.at[ri], bufs.at[slot, 2], sem),
            pltpu.make_async_copy(rel_t.at[ri], bufs.at[slot, 3], sem),
            pltpu.make_async_copy(dots.at[hi], dbufs.at[slot, 0, pl.ds(0, B_C)],
                                  sem),
            pltpu.make_async_copy(dots.at[ti], dbufs.at[slot, 1, pl.ds(0, B_C)],
                                  sem),
        )

    def fire(g, slot):
        for cp in copies(g, slot):
            cp.start()

    def drain(g, slot):
        for cp in copies(g, slot):
            cp.wait()

    def compute(g, slot):
        hb = bufs.at[slot, 0]
        tb = bufs.at[slot, 1]
        rnb = bufs.at[slot, 2]
        rtb = bufs.at[slot, 3]
        dhb = dbufs.at[slot, 0]
        dtb = dbufs.at[slot, 1]
        last_lane = lax.iota(jnp.int32, 16) == 15

        def tri(i):
            h = [hb[i, pl.ds(16 * j, 16)] for j in range(NV)]
            t = [tb[i, pl.ds(16 * j, 16)] for j in range(NV)]
            rn = [rnb[i, pl.ds(16 * j, 16)] for j in range(NV)]
            rt = [rtb[i, pl.ds(16 * j, 16)] for j in range(NV)]
            dhs = dhb[pl.ds(i, 16)][0]
            dts = dtb[pl.ds(i, 16)][0]

            hp = [h[j] + dhs * rt[j] for j in range(NV)]
            tp = [t[j] + dts * rt[j] for j in range(NV)]
            nh = jnp.sum(_tree_sum([hp[j] * hp[j] for j in range(NV)]))
            nt = jnp.sum(_tree_sum([tp[j] * tp[j] for j in range(NV)]))

            inh = _rsqrt_s(jnp.maximum(nh, 1e-12))
            int_ = _rsqrt_s(jnp.maximum(nt, 1e-12))

            s_acc = _tree_sum([jnp.abs(hp[j] * inh + rn[j] - tp[j] * int_)
                               for j in range(NV)])
            cs = plsc.cumsum(s_acc)
            plsc.store_compressed(scores_v.at[pl.ds(g * B_C + i, 16)], cs,
                                  mask=last_lane)

        plsc.parallel_loop(0, B_C, unroll=2)(tri)

    fire(0, 0)
    fire(1, 1)

    def ring(k, _):
        g0 = k * 2
        for b in range(2):
            g = g0 + b
            drain(g, b)
            compute(g, b)

            @pl.when(g + 2 < B_NCHUNK)
            def _():
                fire(g + 2, b)
        return 0

    lax.fori_loop(0, B_NCHUNK // 2, ring, 0)
    pltpu.sync_copy(scores_v.at[pl.ds(0, B_PER_W)], out.at[pl.ds(base, B_PER_W)])


_sc_scores = pl.kernel(
    _sc_body,
    out_type=jax.ShapeDtypeStruct((N2,), jnp.float32),
    mesh=plsc.VectorSubcoreMesh(core_axis_name="c", subcore_axis_name="s"),
    compiler_params=pltpu.CompilerParams(needs_layout_passes=False),
    scratch_types=[
        pltpu.VMEM((B_PER_W,), jnp.int32),
        pltpu.VMEM((B_PER_W,), jnp.int32),
        pltpu.VMEM((B_PER_W,), jnp.int32),
        pltpu.VMEM((2, 4, B_C, D), jnp.float32),
        pltpu.VMEM((2, 2, B_C + 16), jnp.float32),
        pltpu.VMEM((B_PER_W + 16,), jnp.float32),
        pltpu.SemaphoreType.DMA,
        pltpu.SemaphoreType.DMA,
    ],
)


def _tail_body(ps_ref, ns_ref, nss_ref, sim_ref, loss_ref, negloss_ref):
    ps = ps_ref[...]
    ns = ns_ref[...]
    loss_ref[...] = jnp.sum(jnp.maximum(ps - ns + MARGIN, 0.0),
                            axis=(0, 1), keepdims=True)
    nss = nss_ref[...]
    sim = sim_ref[...]
    a = jax.nn.softmax(nss, axis=-1)
    b = jax.nn.softmax(sim, axis=-1)
    negloss_ref[...] = jnp.sum(a * b, axis=(0, 1), keepdims=True) / BATCH


def _tail(ps, ns, nss, sim):
    return pl.pallas_call(
        _tail_body,
        out_shape=[
            jax.ShapeDtypeStruct((1, 1), jnp.float32),
            jax.ShapeDtypeStruct((1, 1), jnp.float32),
        ],
    )(ps, ns, nss, sim)


def kernel(ent_embeddings, rel_embeddings, ent_transfer, rel_transfer,
           pos_h, pos_t, pos_r, neg_h, neg_t, neg_r,
           neg_hs, neg_ts, neg_rs, neg_sim):
    i32 = jnp.int32
    H = jnp.concatenate([pos_h.astype(i32), neg_h.astype(i32),
                         neg_hs.astype(i32).reshape(-1)])
    T = jnp.concatenate([pos_t.astype(i32), neg_t.astype(i32),
                         neg_ts.astype(i32).reshape(-1)])
    R = jnp.concatenate([pos_r.astype(i32), neg_r.astype(i32),
                         neg_rs.astype(i32).reshape(-1)])
    dots = _ent_dots(ent_embeddings, ent_transfer)
    rel_n = _rel_norm(rel_embeddings)
    scores_a = _sc_scores_a(ent_embeddings, rel_embeddings, ent_transfer,
                            rel_transfer, H[:N1], T[:N1], R[:N1])
    scores_b = _sc_scores(ent_embeddings, rel_n, rel_transfer, dots,
                          H[N1:], T[N1:], R[N1:])
    ps = scores_a[:BATCH]
    ns = scores_a[BATCH:2 * BATCH]
    nss = jnp.concatenate([scores_a[2 * BATCH:],
                           scores_b]).reshape(BATCH, NEG_NUM)
    loss2, negloss2 = _tail(ps.reshape(BATCH // D, D),
                            ns.reshape(BATCH // D, D), nss, neg_sim)
    return loss2[0, 0], negloss2[0, 0], ps
